# Initial kernel scaffold; baseline (speedup 1.0000x reference)
import functools

import jax
import jax.numpy as jnp
from jax import lax
from jax.experimental import pallas as pl
from jax.experimental.pallas import tpu as pltpu
from jax.experimental.pallas import tpu_sc as plsc

B = 16384
D = 64

_NC = 2
_NS = 16
_NW = _NC * _NS
_BPW = B // _NW


def _gather6(user_idx, item_idx, t_umlp, t_imlp, t_umf, t_imf, t_uv, t_ua):
    mesh = plsc.VectorSubcoreMesh(core_axis_name="c", subcore_axis_name="s")

    @functools.partial(
        pl.kernel,
        out_type=[jax.ShapeDtypeStruct((B, D), jnp.float32)] * 6,
        mesh=mesh,
        scratch_types=[
            pltpu.VMEM((_BPW,), jnp.int32),
            pltpu.VMEM((_BPW,), jnp.int32),
            pltpu.VMEM((_BPW, D), jnp.float32),
            pltpu.VMEM((_BPW, D), jnp.float32),
            pltpu.SemaphoreType.DMA,
            pltpu.SemaphoreType.DMA,
        ],
    )
    def k(uidx_h, iidx_h, tum, tim, tumf, timf, tuv, tua,
          o_um, o_im, o_umf, o_imf, o_uv, o_ua,
          uv, iv, buf0, buf1, sem0, sem1):
        wid = lax.axis_index("s") * _NC + lax.axis_index("c")
        base = wid * _BPW
        pltpu.sync_copy(uidx_h.at[pl.ds(base, _BPW)], uv)
        pltpu.sync_copy(iidx_h.at[pl.ds(base, _BPW)], iv)
        jobs = [(tum, uv, o_um), (tim, iv, o_im), (tumf, uv, o_umf),
                (timf, iv, o_imf), (tuv, uv, o_uv), (tua, uv, o_ua)]
        bufs = (buf0, buf1)
        sems = (sem0, sem1)
        cps = {}
        for t in (0, 1):
            tab, idxv, _ = jobs[t]
            cps[t] = pltpu.async_copy(tab.at[idxv], bufs[t % 2], sems[t % 2])
        for t in range(6):
            cps[t].wait()
            pltpu.sync_copy(bufs[t % 2], jobs[t][2].at[pl.ds(base, _BPW)])
            if t + 2 < 6:
                tab, idxv, _ = jobs[t + 2]
                cps[t + 2] = pltpu.async_copy(tab.at[idxv], bufs[t % 2],
                                              sems[t % 2])

    return k(user_idx, item_idx, t_umlp, t_imlp, t_umf, t_imf, t_uv, t_ua)


_BLK = 1024


def _dense_body(pb, umlp, imlp, umf, imf, uvv, ua,
                few0, feb0, few1, feb1,
                fcw0u, fcw0i, fcb0, fcw1, fcb1,
                fvw0u, fvw0p, fvb0, fvw1, fvb1,
                atw, atb, afw_mlp, afw_mf, afw_v, afb, out):
    f32 = jnp.float32
    dot = lambda a, b: lax.dot_general(a, b, (((1,), (0,)), ((), ())),
                                       preferred_element_type=f32)
    x = pb[...].astype(jnp.bfloat16)
    h = dot(x, few0[...]) + feb0[...]
    h = jnp.maximum(h, 0.0).astype(jnp.bfloat16)
    pe = dot(h, few1[...]) + feb1[...]

    mf = umf[...] * imf[...]
    mlp = dot(umlp[...], fcw0u[...]) + dot(imlp[...], fcw0i[...]) + fcb0[...]
    mlp = jnp.maximum(mlp, 0.0)
    mlp = jnp.maximum(dot(mlp, fcw1[...]) + fcb1[...], 0.0)

    v = dot(uvv[...], fvw0u[...]) + dot(pe, fvw0p[...]) + fvb0[...]
    v = jnp.maximum(v, 0.0)
    v = jnp.maximum(dot(v, fvw1[...]) + fvb1[...], 0.0)

    att = jax.nn.sigmoid(dot(jnp.maximum(ua[...], 0.0), atw[...]) + atb[...])
    pre = (dot(mlp * att[:, 0:1], afw_mlp[...])
           + dot(mf * att[:, 1:2], afw_mf[...])
           + dot(v * att[:, 2:3], afw_v[...])
           + afb[...])
    out[...] = jax.nn.sigmoid(pre)


def _dense(poster, umlp, imlp, umf, imf, uvv, ua,
           few0, feb0, few1, feb1,
           fcw0u, fcw0i, fcb0, fcw1, fcb1,
           fvw0u, fvw0p, fvb0, fvw1, fvb1,
           atw, atb, afw_mlp, afw_mf, afw_v, afb):
    grid = (B // _BLK,)
    row_spec = lambda w: pl.BlockSpec((_BLK, w), lambda i: (i, 0))
    full = lambda a: pl.BlockSpec(a.shape, lambda i: (0,) * a.ndim)
    args = (poster, umlp, imlp, umf, imf, uvv, ua,
            few0, feb0, few1, feb1,
            fcw0u, fcw0i, fcb0, fcw1, fcb1,
            fvw0u, fvw0p, fvb0, fvw1, fvb1,
            atw, atb, afw_mlp, afw_mf, afw_v, afb)
    in_specs = [row_spec(2048)] + [row_spec(D)] * 6 + [full(a) for a in args[7:]]
    return pl.pallas_call(
        _dense_body,
        grid=grid,
        in_specs=in_specs,
        out_specs=pl.BlockSpec((_BLK, 1), lambda i: (i, 0)),
        out_shape=jax.ShapeDtypeStruct((B, 1), jnp.float32),
    )(*args)


def kernel(user_indices, item_indices, poster_embeddings, emb_user_mlp,
           emb_item_mlp, emb_user_mf, emb_item_mf, emb_user_v, emb_atten,
           fe_W0, fe_b0, fe_W1, fe_b1, fc_W0, fc_b0, fc_W1, fc_b1,
           fv_W0, fv_b0, fv_W1, fv_b1, at_W, at_b, af_W, af_b):
    g_umlp, g_imlp, g_umf, g_imf, g_uv, g_ua = _gather6(
        user_indices, item_indices, emb_user_mlp, emb_item_mlp,
        emb_user_mf, emb_item_mf, emb_user_v, emb_atten)
    bf16 = jnp.bfloat16
    return _dense(
        poster_embeddings, g_umlp, g_imlp, g_umf, g_imf, g_uv, g_ua,
        fe_W0.astype(bf16), fe_b0.reshape(1, -1), fe_W1.astype(bf16),
        fe_b1.reshape(1, -1),
        fc_W0[:64], fc_W0[64:], fc_b0.reshape(1, -1), fc_W1,
        fc_b1.reshape(1, -1),
        fv_W0[:64], fv_W0[64:], fv_b0.reshape(1, -1), fv_W1,
        fv_b1.reshape(1, -1),
        at_W, at_b.reshape(1, -1),
        af_W[:32], af_W[32:96], af_W[96:], af_b.reshape(1, -1))


# speedup vs baseline: 1.0609x; 1.0609x over previous
import functools

import jax
import jax.numpy as jnp
from jax import lax
from jax.experimental import pallas as pl
from jax.experimental.pallas import tpu as pltpu
from jax.experimental.pallas import tpu_sc as plsc

B = 16384
D = 64

_NC = 2
_NS = 16
_NW = _NC * _NS
_BPW = B // _NW


def _gather6(user_idx, item_idx, t_umlp, t_imlp, t_umf, t_imf, t_uv, t_ua):
    mesh = plsc.VectorSubcoreMesh(core_axis_name="c", subcore_axis_name="s")

    @functools.partial(
        pl.kernel,
        out_type=[jax.ShapeDtypeStruct((B, D), jnp.float32)] * 6,
        mesh=mesh,
        compiler_params=pltpu.CompilerParams(use_tc_tiling_on_sc=False),
        scratch_types=[
            pltpu.VMEM((_BPW,), jnp.int32),
            pltpu.VMEM((_BPW,), jnp.int32),
            pltpu.VMEM((_BPW, D), jnp.float32),
            pltpu.VMEM((_BPW, D), jnp.float32),
            pltpu.SemaphoreType.DMA,
            pltpu.SemaphoreType.DMA,
        ],
    )
    def k(uidx_h, iidx_h, tum, tim, tumf, timf, tuv, tua,
          o_um, o_im, o_umf, o_imf, o_uv, o_ua,
          uv, iv, buf0, buf1, sem0, sem1):
        wid = lax.axis_index("s") * _NC + lax.axis_index("c")
        base = wid * _BPW
        pltpu.sync_copy(uidx_h.at[pl.ds(base, _BPW)], uv)
        pltpu.sync_copy(iidx_h.at[pl.ds(base, _BPW)], iv)
        jobs = [(tum, uv, o_um), (tim, iv, o_im), (tumf, uv, o_umf),
                (timf, iv, o_imf), (tuv, uv, o_uv), (tua, uv, o_ua)]
        bufs = (buf0, buf1)
        sems = (sem0, sem1)
        cps = {}
        for t in (0, 1):
            tab, idxv, _ = jobs[t]
            cps[t] = pltpu.async_copy(tab.at[idxv], bufs[t % 2], sems[t % 2])
        for t in range(6):
            cps[t].wait()
            pltpu.sync_copy(bufs[t % 2], jobs[t][2].at[pl.ds(base, _BPW)])
            if t + 2 < 6:
                tab, idxv, _ = jobs[t + 2]
                cps[t + 2] = pltpu.async_copy(tab.at[idxv], bufs[t % 2],
                                              sems[t % 2])

    return k(user_idx, item_idx, t_umlp, t_imlp, t_umf, t_imf, t_uv, t_ua)


_BLK = 1024


def _dense_body(pb, umlp, imlp, umf, imf, uvv, ua,
                few0, feb0, few1, feb1,
                fcw0u, fcw0i, fcb0, fcw1, fcb1,
                fvw0u, fvw0p, fvb0, fvw1, fvb1,
                atw, atb, afw_mlp, afw_mf, afw_v, afb, out):
    f32 = jnp.float32
    dot = lambda a, b: lax.dot_general(a, b, (((1,), (0,)), ((), ())),
                                       preferred_element_type=f32)
    x = pb[...].astype(jnp.bfloat16)
    h = dot(x, few0[...]) + feb0[...]
    h = jnp.maximum(h, 0.0).astype(jnp.bfloat16)
    pe = dot(h, few1[...]) + feb1[...]

    mf = umf[...] * imf[...]
    mlp = dot(umlp[...], fcw0u[...]) + dot(imlp[...], fcw0i[...]) + fcb0[...]
    mlp = jnp.maximum(mlp, 0.0)
    mlp = jnp.maximum(dot(mlp, fcw1[...]) + fcb1[...], 0.0)

    v = dot(uvv[...], fvw0u[...]) + dot(pe, fvw0p[...]) + fvb0[...]
    v = jnp.maximum(v, 0.0)
    v = jnp.maximum(dot(v, fvw1[...]) + fvb1[...], 0.0)

    att = jax.nn.sigmoid(dot(jnp.maximum(ua[...], 0.0), atw[...]) + atb[...])
    pre = (dot(mlp * att[:, 0:1], afw_mlp[...])
           + dot(mf * att[:, 1:2], afw_mf[...])
           + dot(v * att[:, 2:3], afw_v[...])
           + afb[...])
    out[...] = jax.nn.sigmoid(pre)


def _dense(poster, umlp, imlp, umf, imf, uvv, ua,
           few0, feb0, few1, feb1,
           fcw0u, fcw0i, fcb0, fcw1, fcb1,
           fvw0u, fvw0p, fvb0, fvw1, fvb1,
           atw, atb, afw_mlp, afw_mf, afw_v, afb):
    grid = (B // _BLK,)
    row_spec = lambda w: pl.BlockSpec((_BLK, w), lambda i: (i, 0))
    full = lambda a: pl.BlockSpec(a.shape, lambda i: (0,) * a.ndim)
    args = (poster, umlp, imlp, umf, imf, uvv, ua,
            few0, feb0, few1, feb1,
            fcw0u, fcw0i, fcb0, fcw1, fcb1,
            fvw0u, fvw0p, fvb0, fvw1, fvb1,
            atw, atb, afw_mlp, afw_mf, afw_v, afb)
    in_specs = [row_spec(2048)] + [row_spec(D)] * 6 + [full(a) for a in args[7:]]
    return pl.pallas_call(
        _dense_body,
        grid=grid,
        in_specs=in_specs,
        out_specs=pl.BlockSpec((_BLK, 1), lambda i: (i, 0)),
        out_shape=jax.ShapeDtypeStruct((B, 1), jnp.float32),
    )(*args)


def kernel(user_indices, item_indices, poster_embeddings, emb_user_mlp,
           emb_item_mlp, emb_user_mf, emb_item_mf, emb_user_v, emb_atten,
           fe_W0, fe_b0, fe_W1, fe_b1, fc_W0, fc_b0, fc_W1, fc_b1,
           fv_W0, fv_b0, fv_W1, fv_b1, at_W, at_b, af_W, af_b):
    g_umlp, g_imlp, g_umf, g_imf, g_uv, g_ua = _gather6(
        user_indices, item_indices, emb_user_mlp, emb_item_mlp,
        emb_user_mf, emb_item_mf, emb_user_v, emb_atten)
    bf16 = jnp.bfloat16
    return _dense(
        poster_embeddings, g_umlp, g_imlp, g_umf, g_imf, g_uv, g_ua,
        fe_W0.astype(bf16), fe_b0.reshape(1, -1), fe_W1.astype(bf16),
        fe_b1.reshape(1, -1),
        fc_W0[:64], fc_W0[64:], fc_b0.reshape(1, -1), fc_W1,
        fc_b1.reshape(1, -1),
        fv_W0[:64], fv_W0[64:], fv_b0.reshape(1, -1), fv_W1,
        fv_b1.reshape(1, -1),
        at_W, at_b.reshape(1, -1),
        af_W[:32], af_W[32:96], af_W[96:], af_b.reshape(1, -1))
